# baseline (speedup 1.0000x reference)
import functools

import jax
import jax.numpy as jnp
from jax import lax
from jax.experimental import pallas as pl
from jax.experimental.pallas import tpu as pltpu
from jax.experimental.pallas import tpu_sc as plsc

N = 10000
E = 320000
D = 128
H = 128
C = 10
G = 64

NC, NS, L = 2, 16, 16
NW = NC * NS
NPAD = 10240
RPT = NPAD // NS
CHUNK = 128
NCHUNK = 80
EPW = NCHUNK * CHUNK
EPAD = NW * EPW
PAD_ROW = N
NBUF = 4

_mesh = plsc.VectorSubcoreMesh(core_axis_name="c", subcore_axis_name="s",
                               num_cores=NC, num_subcores=NS)


@functools.partial(
    pl.kernel,
    out_type=jax.ShapeDtypeStruct((NC * NPAD,), jnp.float32),
    mesh=_mesh,
    scratch_types=[
        pltpu.VMEM((NCHUNK, CHUNK), jnp.int32),
        pltpu.VMEM((CHUNK,), jnp.float32),
        pltpu.VMEM((RPT,), jnp.float32),
        pltpu.SemaphoreType.DMA,
        pltpu.VMEM_SHARED((NPAD,), jnp.float32),
    ],
)
def _deg_kernel(dst_hbm, out_hbm, dsts, onesv, zv, dsem, acc):
    cid = lax.axis_index("c")
    sid = lax.axis_index("s")
    wid = cid * NS + sid
    for j in range(CHUNK // L):
        onesv[pl.ds(j * L, L)] = jnp.ones((L,), jnp.float32)

    def zb(i, c):
        zv[pl.ds(i * L, L)] = jnp.zeros((L,), jnp.float32)
        return c

    lax.fori_loop(0, RPT // L, zb, 0)
    pltpu.sync_copy(zv, acc.at[pl.ds(sid * RPT, RPT)])
    pltpu.sync_copy(dst_hbm.at[wid], dsts)
    plsc.subcore_barrier()

    def body(i, c):
        pltpu.async_copy(onesv, acc.at[dsts.at[i]], dsem, add=True)
        return c

    lax.fori_loop(0, NCHUNK, body, 0)

    def drain(i, c):
        pltpu.make_async_copy(out_hbm.at[pl.ds(0, CHUNK)], onesv, dsem).wait()
        return c

    lax.fori_loop(0, NCHUNK, drain, 0)
    plsc.subcore_barrier()
    pltpu.sync_copy(acc.at[pl.ds(sid * RPT, RPT)],
                    out_hbm.at[pl.ds(cid * NPAD + sid * RPT, RPT)])


NIB = 4


@functools.partial(
    pl.kernel,
    out_type=jax.ShapeDtypeStruct((NC * NPAD, H), jnp.float32),
    mesh=_mesh,
    scratch_types=[
        pltpu.VMEM((NIB, CHUNK), jnp.int32),
        pltpu.VMEM((NIB, CHUNK), jnp.int32),
        [pltpu.VMEM((CHUNK, H), jnp.float32)] * 2,
        pltpu.VMEM((64, H), jnp.float32),
        [pltpu.SemaphoreType.DMA] * NIB,
        [pltpu.SemaphoreType.DMA] * 2,
        pltpu.SemaphoreType.DMA,
        pltpu.VMEM_SHARED((NPAD, H), jnp.float32),
    ],
)
def _edge_aggregate(h_hbm, src_hbm, dst_hbm, out_hbm,
                    isrc, idst, rows, zrows, isem, gsem, zsem, acc):
    cid = lax.axis_index("c")
    sid = lax.axis_index("s")
    wid = cid * NS + sid

    def _fire_idx(k, q):
        pltpu.async_copy(src_hbm.at[wid, k], isrc.at[q], isem[q])
        pltpu.async_copy(dst_hbm.at[wid, k], idst.at[q], isem[q])

    def _wait_idx(q):
        for _ in range(2):
            pltpu.make_async_copy(src_hbm.at[0, 0], isrc.at[q],
                                  isem[q]).wait()

    def _wait_rows(sem):
        pltpu.make_async_copy(h_hbm.at[pl.ds(0, CHUNK)], rows[0], sem).wait()

    def zb(i, c):
        for j in range(H // L):
            zrows[i, pl.ds(j * L, L)] = jnp.zeros((L,), jnp.float32)
        return c

    lax.fori_loop(0, 64, zb, 0)
    for t in range(RPT // 64):
        pltpu.async_copy(zrows, acc.at[pl.ds(sid * RPT + t * 64, 64)], zsem)
    for q in range(NIB):
        _fire_idx(q, q)
    for t in range(RPT // 64):
        pltpu.make_async_copy(h_hbm.at[pl.ds(0, 64)], zrows, zsem).wait()
    plsc.subcore_barrier()

    _wait_idx(0)
    pltpu.async_copy(h_hbm.at[isrc.at[0]], rows[0], gsem[0])

    def outer(i, c):
        for u in range(NIB):
            j = i * NIB + u
            q = u
            b = u % 2
            ob = 1 - b
            qn = (u + 1) % NIB
            qf = (u + 3) % NIB

            if u == 0:
                @pl.when(i >= 1)
                def _():
                    _fire_idx(j + 3, qf)
            else:
                @pl.when(j + 3 < NCHUNK)
                def _():
                    _fire_idx(j + 3, qf)

            _wait_rows(gsem[b])

            def _issue_gather():
                _wait_idx(qn)
                pltpu.async_copy(h_hbm.at[isrc.at[qn]], rows[ob], gsem[ob])

            if u < NIB - 1:
                _issue_gather()
            else:
                @pl.when(j + 1 < NCHUNK)
                def _():
                    _issue_gather()

            pltpu.sync_copy(rows[b], acc.at[idst.at[q]], add=True)
        return c

    lax.fori_loop(0, NCHUNK // NIB, outer, 0)
    plsc.subcore_barrier()
    pltpu.sync_copy(acc.at[pl.ds(sid * RPT, RPT)],
                    out_hbm.at[pl.ds(cid * NPAD + sid * RPT, RPT)])


def _stage1_body(d0, d1, x, w1, dinv_out, h1p_out):
    deg = d0[...] + d1[...] + 1.0
    dinv = lax.rsqrt(deg)
    dinv_out[...] = dinv
    h1p_out[...] = jnp.dot(x[...], w1[...],
                           preferred_element_type=jnp.float32) * dinv


_stage1 = pl.pallas_call(
    _stage1_body,
    out_shape=[jax.ShapeDtypeStruct((NPAD, 1), jnp.float32),
               jax.ShapeDtypeStruct((NPAD, H), jnp.float32)],
)


def _stage2_body(a0, a1, h1p, dinv, b1, w2, h2p_out):
    dv = dinv[...]
    z = (a0[...] + a1[...] + h1p[...]) * dv + b1[...]
    z = jnp.maximum(z, 0.0)
    h2p_out[...] = jnp.dot(z, w2[...],
                           preferred_element_type=jnp.float32) * dv


_stage2 = pl.pallas_call(
    _stage2_body,
    out_shape=jax.ShapeDtypeStruct((NPAD, H), jnp.float32),
)


def _stage3_body(a0, a1, h2p, dinv, b2, batch8, wl, bl, out):
    z = (a0[...] + a1[...] + h2p[...]) * dinv[...] + b2[...]
    ids = batch8[0:1, :]
    seg = lax.broadcasted_iota(jnp.int32, (G, NPAD), 0)
    oht = (seg == ids).astype(jnp.float32)
    sums = jnp.dot(oht, z, preferred_element_type=jnp.float32)
    counts = jnp.sum(oht, axis=1, keepdims=True)
    pooled = sums / jnp.maximum(counts, 1.0)
    out[...] = jnp.dot(pooled, wl[...],
                       preferred_element_type=jnp.float32) + bl[...]


_stage3 = pl.pallas_call(
    _stage3_body,
    out_shape=jax.ShapeDtypeStruct((G, C), jnp.float32),
)


def kernel(x, edge_index, batch, W1, b1, W2, b2, Wl, bl):
    f32 = jnp.float32
    src = (jnp.full((EPAD,), PAD_ROW, jnp.int32).at[:E].set(edge_index[0])
           .reshape(NW, NCHUNK, CHUNK))
    dst = (jnp.full((EPAD,), PAD_ROW, jnp.int32).at[:E].set(edge_index[1])
           .reshape(NW, NCHUNK, CHUNK))
    xp = jnp.zeros((NPAD, D), f32).at[:N].set(x)
    bpad = jnp.pad(batch.astype(jnp.int32), (0, NPAD - N), constant_values=G)
    batch8 = jnp.broadcast_to(bpad[None, :], (8, NPAD))

    degp = _deg_kernel(dst)
    d0 = degp[:NPAD].reshape(NPAD, 1)
    d1 = degp[NPAD:].reshape(NPAD, 1)

    dinv, h1p = _stage1(d0, d1, xp, W1)
    acc1 = _edge_aggregate(h1p, src, dst)
    h2p = _stage2(acc1[:NPAD], acc1[NPAD:], h1p, dinv,
                  b1.reshape(1, H), W2)
    acc2 = _edge_aggregate(h2p, src, dst)
    out = _stage3(acc2[:NPAD], acc2[NPAD:], h2p, dinv,
                  b2.reshape(1, H), batch8, Wl, bl.reshape(1, C))
    return out


# speedup vs baseline: 1.0124x; 1.0124x over previous
import functools

import jax
import jax.numpy as jnp
from jax import lax
from jax.experimental import pallas as pl
from jax.experimental.pallas import tpu as pltpu
from jax.experimental.pallas import tpu_sc as plsc

N = 10000
E = 320000
D = 128
H = 128
C = 10
G = 64

NC, NS, L = 2, 16, 16
NW = NC * NS
NPAD = 10240
FPT = H // NW
PAD_ROW = N

DCHUNK = 128
DNCH = 80
EPW = DNCH * DCHUNK
EPAD = NW * EPW

ECHUNK = 1024
ENCH = EPAD // ECHUNK
NGRP = ECHUNK // L
ESPT = EPAD // NS

_mesh = plsc.VectorSubcoreMesh(core_axis_name="c", subcore_axis_name="s",
                               num_cores=NC, num_subcores=NS)


@functools.partial(
    pl.kernel,
    out_type=jax.ShapeDtypeStruct((NC * NPAD,), jnp.float32),
    mesh=_mesh,
    scratch_types=[
        pltpu.VMEM((DNCH, DCHUNK), jnp.int32),
        pltpu.VMEM((DCHUNK,), jnp.float32),
        pltpu.VMEM((NPAD // NS,), jnp.float32),
        pltpu.SemaphoreType.DMA,
        pltpu.VMEM_SHARED((NPAD,), jnp.float32),
    ],
)
def _deg_kernel(dst_hbm, out_hbm, dsts, onesv, zv, dsem, acc):
    cid = lax.axis_index("c")
    sid = lax.axis_index("s")
    wid = cid * NS + sid
    rpt = NPAD // NS
    for j in range(DCHUNK // L):
        onesv[pl.ds(j * L, L)] = jnp.ones((L,), jnp.float32)

    def zb(i, c):
        zv[pl.ds(i * L, L)] = jnp.zeros((L,), jnp.float32)
        return c

    lax.fori_loop(0, rpt // L, zb, 0)
    pltpu.sync_copy(zv, acc.at[pl.ds(sid * rpt, rpt)])
    pltpu.sync_copy(dst_hbm.at[wid], dsts)
    plsc.subcore_barrier()

    def body(i, c):
        pltpu.async_copy(onesv, acc.at[dsts.at[i]], dsem, add=True)
        return c

    lax.fori_loop(0, DNCH, body, 0)

    def drain(i, c):
        pltpu.make_async_copy(out_hbm.at[pl.ds(0, DCHUNK)], onesv,
                              dsem).wait()
        return c

    lax.fori_loop(0, DNCH, drain, 0)
    plsc.subcore_barrier()
    pltpu.sync_copy(acc.at[pl.ds(sid * rpt, rpt)],
                    out_hbm.at[pl.ds(cid * NPAD + sid * rpt, rpt)])


@functools.partial(
    pl.kernel,
    out_type=jax.ShapeDtypeStruct((H * NPAD,), jnp.float32),
    mesh=_mesh,
    scratch_types=[
        pltpu.VMEM((FPT * NPAD,), jnp.float32),
        pltpu.VMEM((FPT * NPAD,), jnp.float32),
        [pltpu.VMEM((ECHUNK,), jnp.int32)] * 2,
        [pltpu.VMEM((ECHUNK,), jnp.int32)] * 2,
        [pltpu.SemaphoreType.DMA] * 2,
        pltpu.VMEM_SHARED((EPAD,), jnp.int32),
        pltpu.VMEM_SHARED((EPAD,), jnp.int32),
    ],
    compiler_params=pltpu.CompilerParams(needs_layout_passes=False),
)
def _edge_aggregate(ht_hbm, src_hbm, dst_hbm, out_hbm,
                    hloc, acc, ibs, ibd, isem, ssrc, sdst):
    cid = lax.axis_index("c")
    sid = lax.axis_index("s")
    fbase = (cid * NS + sid) * FPT * NPAD

    pltpu.sync_copy(src_hbm.at[pl.ds(sid * ESPT, ESPT)],
                    ssrc.at[pl.ds(sid * ESPT, ESPT)])
    pltpu.sync_copy(dst_hbm.at[pl.ds(sid * ESPT, ESPT)],
                    sdst.at[pl.ds(sid * ESPT, ESPT)])
    pltpu.sync_copy(ht_hbm.at[pl.ds(fbase, FPT * NPAD)], hloc)

    def zb(i, c):
        acc[pl.ds(i * L, L)] = jnp.zeros((L,), jnp.float32)
        return c

    lax.fori_loop(0, FPT * NPAD // L, zb, 0)
    plsc.subcore_barrier()

    def _fire_idx(k, r):
        pltpu.async_copy(ssrc.at[pl.ds(k * ECHUNK, ECHUNK)], ibs[r], isem[r])
        pltpu.async_copy(sdst.at[pl.ds(k * ECHUNK, ECHUNK)], ibd[r], isem[r])

    def _wait_idx(r):
        for _ in range(2):
            pltpu.make_async_copy(src_hbm.at[pl.ds(0, ECHUNK)], ibs[r],
                                  isem[r]).wait()

    _fire_idx(0, 0)

    fofs = [jnp.full((L,), f * NPAD, jnp.int32) for f in range(FPT)]

    def chunk_body(k, c):
        for r in range(2):
            @pl.when(k % 2 == r)
            def _():
                _wait_idx(r)

                @pl.when(k + 1 < ENCH)
                def _():
                    _fire_idx(k + 1, 1 - r)

                def grp(g, cc):
                    srcv = ibs[r][pl.ds(g * L, L)]
                    dstv = ibd[r][pl.ds(g * L, L)]
                    for f in range(FPT):
                        v = plsc.load_gather(hloc, [srcv + fofs[f]])
                        plsc.addupdate_scatter(acc, [dstv + fofs[f]], v)
                    return cc

                lax.fori_loop(0, NGRP, grp, 0)
        return c

    lax.fori_loop(0, ENCH, chunk_body, 0)
    pltpu.sync_copy(acc, out_hbm.at[pl.ds(fbase, FPT * NPAD)])


def _stage1_body(degp, xt, w1t, dinv_out, h1t_out):
    deg = degp[0:1, :] + degp[1:2, :] + 1.0
    dinv = lax.rsqrt(deg)
    dinv_out[...] = jnp.broadcast_to(dinv, (8, NPAD))
    h1t_out[...] = jnp.dot(w1t[...], xt[...],
                           preferred_element_type=jnp.float32) * dinv


_stage1 = pl.pallas_call(
    _stage1_body,
    out_shape=[jax.ShapeDtypeStruct((8, NPAD), jnp.float32),
               jax.ShapeDtypeStruct((H, NPAD), jnp.float32)],
)


def _stage2_body(acct, h1t, dinv8, b1c, w2t, h2t_out):
    dinv = dinv8[0:1, :]
    z = (acct[...] + h1t[...]) * dinv + b1c[...]
    z = jnp.maximum(z, 0.0)
    h2t_out[...] = jnp.dot(w2t[...], z,
                           preferred_element_type=jnp.float32) * dinv


_stage2 = pl.pallas_call(
    _stage2_body,
    out_shape=jax.ShapeDtypeStruct((H, NPAD), jnp.float32),
)


def _stage3_body(acct, h2t, dinv8, b2c, batch8, wl, bl, out):
    z = (acct[...] + h2t[...]) * dinv8[0:1, :] + b2c[...]
    ids = batch8[0:1, :]
    seg = lax.broadcasted_iota(jnp.int32, (G, NPAD), 0)
    oht = (seg == ids).astype(jnp.float32)
    sums = lax.dot_general(oht, z, (((1,), (1,)), ((), ())),
                           preferred_element_type=jnp.float32)
    counts = jnp.sum(oht, axis=1, keepdims=True)
    pooled = sums / jnp.maximum(counts, 1.0)
    out[...] = jnp.dot(pooled, wl[...],
                       preferred_element_type=jnp.float32) + bl[...]


_stage3 = pl.pallas_call(
    _stage3_body,
    out_shape=jax.ShapeDtypeStruct((G, C), jnp.float32),
)


def kernel(x, edge_index, batch, W1, b1, W2, b2, Wl, bl):
    f32 = jnp.float32
    src = jnp.full((EPAD,), PAD_ROW, jnp.int32).at[:E].set(edge_index[0])
    dst = jnp.full((EPAD,), PAD_ROW, jnp.int32).at[:E].set(edge_index[1])
    dst_slab = dst.reshape(NW, DNCH, DCHUNK)
    xt = jnp.zeros((D, NPAD), f32).at[:, :N].set(x.T)
    bpad = jnp.pad(batch.astype(jnp.int32), (0, NPAD - N), constant_values=G)
    batch8 = jnp.broadcast_to(bpad[None, :], (8, NPAD))

    degp = _deg_kernel(dst_slab)
    degp8 = jnp.zeros((8, NPAD), f32).at[:2].set(degp.reshape(2, NPAD))

    dinv8, h1t = _stage1(degp8, xt, W1.T)
    acc1 = _edge_aggregate(h1t.reshape(H * NPAD), src, dst).reshape(H, NPAD)
    h2t = _stage2(acc1, h1t, dinv8, b1.reshape(H, 1), W2.T)
    acc2 = _edge_aggregate(h2t.reshape(H * NPAD), src, dst).reshape(H, NPAD)
    out = _stage3(acc2, h2t, dinv8, b2.reshape(H, 1), batch8,
                  Wl, bl.reshape(1, C))
    return out


# speedup vs baseline: 1.8640x; 1.8412x over previous
import functools

import jax
import jax.numpy as jnp
from jax import lax
from jax.experimental import pallas as pl
from jax.experimental.pallas import tpu as pltpu
from jax.experimental.pallas import tpu_sc as plsc

N = 10000
E = 320000
D = 128
H = 128
C = 10
G = 64

NC, NS, L = 2, 16, 16
NW = NC * NS
NPAD = 10240
FPT = H // NW
PAD_ROW = N

DCHUNK = 128
DNCH = 80
EPW = DNCH * DCHUNK
EPAD = NW * EPW

ECHUNK = 1024
ENCH = EPAD // ECHUNK
NGRP = ECHUNK // L
ESPT = EPAD // NS

_mesh = plsc.VectorSubcoreMesh(core_axis_name="c", subcore_axis_name="s",
                               num_cores=NC, num_subcores=NS)


@functools.partial(
    pl.kernel,
    out_type=jax.ShapeDtypeStruct((NC * NPAD,), jnp.float32),
    mesh=_mesh,
    scratch_types=[
        pltpu.VMEM((DNCH, DCHUNK), jnp.int32),
        pltpu.VMEM((DCHUNK,), jnp.float32),
        pltpu.VMEM((NPAD // NS,), jnp.float32),
        pltpu.SemaphoreType.DMA,
        pltpu.VMEM_SHARED((NPAD,), jnp.float32),
    ],
)
def _deg_kernel(dst_hbm, out_hbm, dsts, onesv, zv, dsem, acc):
    cid = lax.axis_index("c")
    sid = lax.axis_index("s")
    wid = cid * NS + sid
    rpt = NPAD // NS
    for j in range(DCHUNK // L):
        onesv[pl.ds(j * L, L)] = jnp.ones((L,), jnp.float32)

    def zb(i, c):
        zv[pl.ds(i * L, L)] = jnp.zeros((L,), jnp.float32)
        return c

    lax.fori_loop(0, rpt // L, zb, 0)
    pltpu.sync_copy(zv, acc.at[pl.ds(sid * rpt, rpt)])
    pltpu.sync_copy(dst_hbm.at[wid], dsts)
    plsc.subcore_barrier()

    def body(i, c):
        pltpu.async_copy(onesv, acc.at[dsts.at[i]], dsem, add=True)
        return c

    lax.fori_loop(0, DNCH, body, 0)

    def drain(i, c):
        pltpu.make_async_copy(out_hbm.at[pl.ds(0, DCHUNK)], onesv,
                              dsem).wait()
        return c

    lax.fori_loop(0, DNCH, drain, 0)
    plsc.subcore_barrier()
    pltpu.sync_copy(acc.at[pl.ds(sid * rpt, rpt)],
                    out_hbm.at[pl.ds(cid * NPAD + sid * rpt, rpt)])


@functools.partial(
    pl.kernel,
    out_type=jax.ShapeDtypeStruct((H * NPAD,), jnp.float32),
    mesh=_mesh,
    scratch_types=[
        pltpu.VMEM((FPT * NPAD,), jnp.float32),
        pltpu.VMEM((FPT * NPAD,), jnp.float32),
        [pltpu.VMEM((ECHUNK,), jnp.int32)] * 2,
        [pltpu.VMEM((ECHUNK,), jnp.int32)] * 2,
        [pltpu.SemaphoreType.DMA] * 2,
        pltpu.VMEM_SHARED((EPAD,), jnp.int32),
        pltpu.VMEM_SHARED((EPAD,), jnp.int32),
    ],
    compiler_params=pltpu.CompilerParams(needs_layout_passes=False),
)
def _edge_aggregate(ht_hbm, src_hbm, dst_hbm, out_hbm,
                    hloc, acc, ibs, ibd, isem, ssrc, sdst):
    cid = lax.axis_index("c")
    sid = lax.axis_index("s")
    fbase = (cid * NS + sid) * FPT * NPAD

    pltpu.sync_copy(src_hbm.at[pl.ds(sid * ESPT, ESPT)],
                    ssrc.at[pl.ds(sid * ESPT, ESPT)])
    pltpu.sync_copy(dst_hbm.at[pl.ds(sid * ESPT, ESPT)],
                    sdst.at[pl.ds(sid * ESPT, ESPT)])
    pltpu.sync_copy(ht_hbm.at[pl.ds(fbase, FPT * NPAD)], hloc)

    def zb(i, c):
        acc[pl.ds(i * L, L)] = jnp.zeros((L,), jnp.float32)
        return c

    lax.fori_loop(0, FPT * NPAD // L, zb, 0)
    plsc.subcore_barrier()

    def _fire_idx(k, r):
        pltpu.async_copy(ssrc.at[pl.ds(k * ECHUNK, ECHUNK)], ibs[r], isem[r])
        pltpu.async_copy(sdst.at[pl.ds(k * ECHUNK, ECHUNK)], ibd[r], isem[r])

    def _wait_idx(r):
        for _ in range(2):
            pltpu.make_async_copy(src_hbm.at[pl.ds(0, ECHUNK)], ibs[r],
                                  isem[r]).wait()

    _fire_idx(0, 0)

    fofs = [jnp.full((L,), f * NPAD, jnp.int32) for f in range(FPT)]

    def chunk_body(k, c):
        for r in range(2):
            @pl.when(k % 2 == r)
            def _():
                _wait_idx(r)

                @pl.when(k + 1 < ENCH)
                def _():
                    _fire_idx(k + 1, 1 - r)

                def grp(g, cc):
                    vals = []
                    for u in range(2):
                        srcv = ibs[r][pl.ds((g * 2 + u) * L, L)]
                        dstv = ibd[r][pl.ds((g * 2 + u) * L, L)]
                        vals.append(
                            (dstv,
                             [plsc.load_gather(hloc, [srcv + fofs[f]])
                              for f in range(FPT)]))
                    for dstv, vs in vals:
                        for f in range(FPT):
                            plsc.addupdate_scatter(acc, [dstv + fofs[f]],
                                                   vs[f])
                    return cc

                lax.fori_loop(0, NGRP // 2, grp, 0)
        return c

    lax.fori_loop(0, ENCH, chunk_body, 0)
    pltpu.sync_copy(acc, out_hbm.at[pl.ds(fbase, FPT * NPAD)])


def _stage1_body(degp, xt, w1t, dinv_out, h1t_out):
    deg = degp[0:1, :] + degp[1:2, :] + 1.0
    dinv = lax.rsqrt(deg)
    dinv_out[...] = jnp.broadcast_to(dinv, (8, NPAD))
    h1t_out[...] = jnp.dot(w1t[...], xt[...],
                           preferred_element_type=jnp.float32) * dinv


_stage1 = pl.pallas_call(
    _stage1_body,
    out_shape=[jax.ShapeDtypeStruct((8, NPAD), jnp.float32),
               jax.ShapeDtypeStruct((H, NPAD), jnp.float32)],
)


def _stage2_body(acct, h1t, dinv8, b1c, w2t, h2t_out):
    dinv = dinv8[0:1, :]
    z = (acct[...] + h1t[...]) * dinv + b1c[...]
    z = jnp.maximum(z, 0.0)
    h2t_out[...] = jnp.dot(w2t[...], z,
                           preferred_element_type=jnp.float32) * dinv


_stage2 = pl.pallas_call(
    _stage2_body,
    out_shape=jax.ShapeDtypeStruct((H, NPAD), jnp.float32),
)


def _stage3_body(acct, h2t, dinv8, b2c, batch8, wl, bl, out):
    z = (acct[...] + h2t[...]) * dinv8[0:1, :] + b2c[...]
    ids = batch8[0:1, :]
    seg = lax.broadcasted_iota(jnp.int32, (G, NPAD), 0)
    oht = (seg == ids).astype(jnp.float32)
    sums = lax.dot_general(oht, z, (((1,), (1,)), ((), ())),
                           preferred_element_type=jnp.float32)
    counts = jnp.sum(oht, axis=1, keepdims=True)
    pooled = sums / jnp.maximum(counts, 1.0)
    out[...] = jnp.dot(pooled, wl[...],
                       preferred_element_type=jnp.float32) + bl[...]


_stage3 = pl.pallas_call(
    _stage3_body,
    out_shape=jax.ShapeDtypeStruct((G, C), jnp.float32),
)


def kernel(x, edge_index, batch, W1, b1, W2, b2, Wl, bl):
    f32 = jnp.float32
    src = jnp.full((EPAD,), PAD_ROW, jnp.int32).at[:E].set(edge_index[0])
    dst = jnp.full((EPAD,), PAD_ROW, jnp.int32).at[:E].set(edge_index[1])
    dst_slab = dst.reshape(NW, DNCH, DCHUNK)
    xt = jnp.zeros((D, NPAD), f32).at[:, :N].set(x.T)
    bpad = jnp.pad(batch.astype(jnp.int32), (0, NPAD - N), constant_values=G)
    batch8 = jnp.broadcast_to(bpad[None, :], (8, NPAD))

    degp = _deg_kernel(dst_slab)
    degp8 = jnp.zeros((8, NPAD), f32).at[:2].set(degp.reshape(2, NPAD))

    dinv8, h1t = _stage1(degp8, xt, W1.T)
    acc1 = _edge_aggregate(h1t.reshape(H * NPAD), src, dst).reshape(H, NPAD)
    h2t = _stage2(acc1, h1t, dinv8, b1.reshape(H, 1), W2.T)
    acc2 = _edge_aggregate(h2t.reshape(H * NPAD), src, dst).reshape(H, NPAD)
    out = _stage3(acc2, h2t, dinv8, b2.reshape(H, 1), batch8,
                  Wl, bl.reshape(1, C))
    return out


# speedup vs baseline: 2.0115x; 1.0791x over previous
import functools

import jax
import jax.numpy as jnp
from jax import lax
from jax.experimental import pallas as pl
from jax.experimental.pallas import tpu as pltpu
from jax.experimental.pallas import tpu_sc as plsc

N = 10000
E = 320000
D = 128
H = 128
C = 10
G = 64

NC, NS, L = 2, 16, 16
NW = NC * NS
NPAD = 10240
FPT = H // NW
PAD_ROW = N

DCHUNK = 128
DNCH = 80
EPW = DNCH * DCHUNK
EPAD = NW * EPW

ECHUNK = 1024
ENCH = EPAD // ECHUNK
NGRP = ECHUNK // L
ESPT = EPAD // NS

_mesh = plsc.VectorSubcoreMesh(core_axis_name="c", subcore_axis_name="s",
                               num_cores=NC, num_subcores=NS)


@functools.partial(
    pl.kernel,
    out_type=jax.ShapeDtypeStruct((NC * NPAD,), jnp.float32),
    mesh=_mesh,
    scratch_types=[
        pltpu.VMEM((DNCH, DCHUNK), jnp.int32),
        pltpu.VMEM((DCHUNK,), jnp.float32),
        pltpu.VMEM((NPAD // NS,), jnp.float32),
        pltpu.SemaphoreType.DMA,
        pltpu.VMEM_SHARED((NPAD,), jnp.float32),
    ],
)
def _deg_kernel(dst_hbm, out_hbm, dsts, onesv, zv, dsem, acc):
    cid = lax.axis_index("c")
    sid = lax.axis_index("s")
    wid = cid * NS + sid
    rpt = NPAD // NS
    for j in range(DCHUNK // L):
        onesv[pl.ds(j * L, L)] = jnp.ones((L,), jnp.float32)

    def zb(i, c):
        zv[pl.ds(i * L, L)] = jnp.zeros((L,), jnp.float32)
        return c

    lax.fori_loop(0, rpt // L, zb, 0)
    pltpu.sync_copy(zv, acc.at[pl.ds(sid * rpt, rpt)])
    pltpu.sync_copy(dst_hbm.at[wid], dsts)
    plsc.subcore_barrier()

    def body(i, c):
        pltpu.async_copy(onesv, acc.at[dsts.at[i]], dsem, add=True)
        return c

    lax.fori_loop(0, DNCH, body, 0)

    def drain(i, c):
        pltpu.make_async_copy(out_hbm.at[pl.ds(0, DCHUNK)], onesv,
                              dsem).wait()
        return c

    lax.fori_loop(0, DNCH, drain, 0)
    plsc.subcore_barrier()
    pltpu.sync_copy(acc.at[pl.ds(sid * rpt, rpt)],
                    out_hbm.at[pl.ds(cid * NPAD + sid * rpt, rpt)])


@functools.partial(
    pl.kernel,
    out_type=jax.ShapeDtypeStruct((H * NPAD,), jnp.float32),
    mesh=_mesh,
    scratch_types=[
        pltpu.VMEM((FPT * NPAD,), jnp.float32),
        pltpu.VMEM((FPT * NPAD,), jnp.float32),
        [pltpu.VMEM((ECHUNK,), jnp.int32)] * 2,
        [pltpu.VMEM((ECHUNK,), jnp.int32)] * 2,
        [pltpu.SemaphoreType.DMA] * 2,
        pltpu.VMEM_SHARED((EPAD,), jnp.int32),
        pltpu.VMEM_SHARED((EPAD,), jnp.int32),
    ],
    compiler_params=pltpu.CompilerParams(needs_layout_passes=False),
)
def _edge_aggregate(ht_hbm, src_hbm, dst_hbm, out_hbm,
                    hloc, acc, ibs, ibd, isem, ssrc, sdst):
    cid = lax.axis_index("c")
    sid = lax.axis_index("s")
    fbase = (cid * NS + sid) * FPT * NPAD

    pltpu.sync_copy(src_hbm.at[pl.ds(sid * ESPT, ESPT)],
                    ssrc.at[pl.ds(sid * ESPT, ESPT)])
    pltpu.sync_copy(dst_hbm.at[pl.ds(sid * ESPT, ESPT)],
                    sdst.at[pl.ds(sid * ESPT, ESPT)])
    pltpu.sync_copy(ht_hbm.at[pl.ds(fbase, FPT * NPAD)], hloc)

    def zb(i, c):
        acc[pl.ds(i * L, L)] = jnp.zeros((L,), jnp.float32)
        return c

    lax.fori_loop(0, FPT * NPAD // L, zb, 0)
    plsc.subcore_barrier()

    def _fire_idx(k, r):
        pltpu.async_copy(ssrc.at[pl.ds(k * ECHUNK, ECHUNK)], ibs[r], isem[r])
        pltpu.async_copy(sdst.at[pl.ds(k * ECHUNK, ECHUNK)], ibd[r], isem[r])

    def _wait_idx(r):
        for _ in range(2):
            pltpu.make_async_copy(src_hbm.at[pl.ds(0, ECHUNK)], ibs[r],
                                  isem[r]).wait()

    _fire_idx(0, 0)

    fofs = [jnp.full((L,), f * NPAD, jnp.int32) for f in range(FPT)]

    def chunk_body(k, c):
        for r in range(2):
            @pl.when(k % 2 == r)
            def _():
                _wait_idx(r)

                @pl.when(k + 1 < ENCH)
                def _():
                    _fire_idx(k + 1, 1 - r)

                def grp(g, cc):
                    vals = []
                    for u in range(4):
                        srcv = ibs[r][pl.ds((g * 4 + u) * L, L)]
                        dstv = ibd[r][pl.ds((g * 4 + u) * L, L)]
                        vals.append(
                            (dstv,
                             [plsc.load_gather(hloc, [srcv + fofs[f]])
                              for f in range(FPT)]))
                    for dstv, vs in vals:
                        for f in range(FPT):
                            plsc.addupdate_scatter(acc, [dstv + fofs[f]],
                                                   vs[f])
                    return cc

                lax.fori_loop(0, NGRP // 4, grp, 0)
        return c

    lax.fori_loop(0, ENCH, chunk_body, 0)
    pltpu.sync_copy(acc, out_hbm.at[pl.ds(fbase, FPT * NPAD)])


def _stage1_body(degp, xt, w1t, dinv_out, h1t_out):
    deg = degp[0:1, :] + degp[1:2, :] + 1.0
    dinv = lax.rsqrt(deg)
    dinv_out[...] = jnp.broadcast_to(dinv, (8, NPAD))
    h1t_out[...] = jnp.dot(w1t[...], xt[...],
                           preferred_element_type=jnp.float32) * dinv


_stage1 = pl.pallas_call(
    _stage1_body,
    out_shape=[jax.ShapeDtypeStruct((8, NPAD), jnp.float32),
               jax.ShapeDtypeStruct((H, NPAD), jnp.float32)],
)


def _stage2_body(acct, h1t, dinv8, b1c, w2t, h2t_out):
    dinv = dinv8[0:1, :]
    z = (acct[...] + h1t[...]) * dinv + b1c[...]
    z = jnp.maximum(z, 0.0)
    h2t_out[...] = jnp.dot(w2t[...], z,
                           preferred_element_type=jnp.float32) * dinv


_stage2 = pl.pallas_call(
    _stage2_body,
    out_shape=jax.ShapeDtypeStruct((H, NPAD), jnp.float32),
)


def _stage3_body(acct, h2t, dinv8, b2c, batch8, wl, bl, out):
    z = (acct[...] + h2t[...]) * dinv8[0:1, :] + b2c[...]
    ids = batch8[0:1, :]
    seg = lax.broadcasted_iota(jnp.int32, (G, NPAD), 0)
    oht = (seg == ids).astype(jnp.float32)
    sums = lax.dot_general(oht, z, (((1,), (1,)), ((), ())),
                           preferred_element_type=jnp.float32)
    counts = jnp.sum(oht, axis=1, keepdims=True)
    pooled = sums / jnp.maximum(counts, 1.0)
    out[...] = jnp.dot(pooled, wl[...],
                       preferred_element_type=jnp.float32) + bl[...]


_stage3 = pl.pallas_call(
    _stage3_body,
    out_shape=jax.ShapeDtypeStruct((G, C), jnp.float32),
)


def kernel(x, edge_index, batch, W1, b1, W2, b2, Wl, bl):
    f32 = jnp.float32
    src = jnp.full((EPAD,), PAD_ROW, jnp.int32).at[:E].set(edge_index[0])
    dst = jnp.full((EPAD,), PAD_ROW, jnp.int32).at[:E].set(edge_index[1])
    dst_slab = dst.reshape(NW, DNCH, DCHUNK)
    xt = jnp.zeros((D, NPAD), f32).at[:, :N].set(x.T)
    bpad = jnp.pad(batch.astype(jnp.int32), (0, NPAD - N), constant_values=G)
    batch8 = jnp.broadcast_to(bpad[None, :], (8, NPAD))

    degp = _deg_kernel(dst_slab)
    degp8 = jnp.zeros((8, NPAD), f32).at[:2].set(degp.reshape(2, NPAD))

    dinv8, h1t = _stage1(degp8, xt, W1.T)
    acc1 = _edge_aggregate(h1t.reshape(H * NPAD), src, dst).reshape(H, NPAD)
    h2t = _stage2(acc1, h1t, dinv8, b1.reshape(H, 1), W2.T)
    acc2 = _edge_aggregate(h2t.reshape(H * NPAD), src, dst).reshape(H, NPAD)
    out = _stage3(acc2, h2t, dinv8, b2.reshape(H, 1), batch8,
                  Wl, bl.reshape(1, C))
    return out


# speedup vs baseline: 2.1298x; 1.0588x over previous
import functools

import jax
import jax.numpy as jnp
from jax import lax
from jax.experimental import pallas as pl
from jax.experimental.pallas import tpu as pltpu
from jax.experimental.pallas import tpu_sc as plsc

N = 10000
E = 320000
D = 128
H = 128
C = 10
G = 64

NC, NS, L = 2, 16, 16
NW = NC * NS
NPAD = 10240
FPT = H // NW
PAD_ROW = N

DCHUNK = 128
DNCH = 80
EPW = DNCH * DCHUNK
EPAD = NW * EPW

ECHUNK = 1024
ENCH = EPAD // ECHUNK
NGRP = ECHUNK // L
ESPT = EPAD // NS

_mesh = plsc.VectorSubcoreMesh(core_axis_name="c", subcore_axis_name="s",
                               num_cores=NC, num_subcores=NS)


@functools.partial(
    pl.kernel,
    out_type=jax.ShapeDtypeStruct((NC * NPAD,), jnp.float32),
    mesh=_mesh,
    scratch_types=[
        pltpu.VMEM((DNCH, DCHUNK), jnp.int32),
        pltpu.VMEM((DCHUNK,), jnp.float32),
        pltpu.VMEM((NPAD // NS,), jnp.float32),
        pltpu.SemaphoreType.DMA,
        pltpu.VMEM_SHARED((NPAD,), jnp.float32),
    ],
)
def _deg_kernel(dst_hbm, out_hbm, dsts, onesv, zv, dsem, acc):
    cid = lax.axis_index("c")
    sid = lax.axis_index("s")
    wid = cid * NS + sid
    rpt = NPAD // NS
    for j in range(DCHUNK // L):
        onesv[pl.ds(j * L, L)] = jnp.ones((L,), jnp.float32)

    def zb(i, c):
        zv[pl.ds(i * L, L)] = jnp.zeros((L,), jnp.float32)
        return c

    lax.fori_loop(0, rpt // L, zb, 0)
    pltpu.sync_copy(zv, acc.at[pl.ds(sid * rpt, rpt)])
    pltpu.sync_copy(dst_hbm.at[wid], dsts)
    plsc.subcore_barrier()

    def body(i, c):
        pltpu.async_copy(onesv, acc.at[dsts.at[i]], dsem, add=True)
        return c

    lax.fori_loop(0, DNCH, body, 0)

    def drain(i, c):
        pltpu.make_async_copy(out_hbm.at[pl.ds(0, DCHUNK)], onesv,
                              dsem).wait()
        return c

    lax.fori_loop(0, DNCH, drain, 0)
    plsc.subcore_barrier()
    pltpu.sync_copy(acc.at[pl.ds(sid * rpt, rpt)],
                    out_hbm.at[pl.ds(cid * NPAD + sid * rpt, rpt)])


@functools.partial(
    pl.kernel,
    out_type=jax.ShapeDtypeStruct((H * NPAD,), jnp.float32),
    mesh=_mesh,
    scratch_types=[
        pltpu.VMEM((FPT // 2 * NPAD,), jnp.float32),
        pltpu.VMEM((FPT * NPAD,), jnp.float32),
        [pltpu.VMEM((ECHUNK,), jnp.int32)] * 2,
        [pltpu.VMEM((ECHUNK,), jnp.int32)] * 2,
        [pltpu.SemaphoreType.DMA] * 2,
        pltpu.VMEM_SHARED((EPAD,), jnp.int32),
        pltpu.VMEM_SHARED((EPAD,), jnp.int32),
    ],
    compiler_params=pltpu.CompilerParams(needs_layout_passes=False),
)
def _edge_aggregate(ht_hbm, src_hbm, dst_hbm, out_hbm,
                    hloc, acc, ibs, ibd, isem, ssrc, sdst):
    cid = lax.axis_index("c")
    sid = lax.axis_index("s")
    wid = cid * NS + sid
    fbase = wid * FPT * NPAD
    pbase = wid * (FPT // 2) * NPAD

    pltpu.sync_copy(src_hbm.at[pl.ds(sid * ESPT, ESPT)],
                    ssrc.at[pl.ds(sid * ESPT, ESPT)])
    pltpu.sync_copy(dst_hbm.at[pl.ds(sid * ESPT, ESPT)],
                    sdst.at[pl.ds(sid * ESPT, ESPT)])
    pltpu.sync_copy(ht_hbm.at[pl.ds(pbase, FPT // 2 * NPAD)], hloc)

    def zb(i, c):
        acc[pl.ds(i * L, L)] = jnp.zeros((L,), jnp.float32)
        return c

    lax.fori_loop(0, FPT * NPAD // L, zb, 0)
    plsc.subcore_barrier()

    def _fire_idx(k, r):
        pltpu.async_copy(ssrc.at[pl.ds(k * ECHUNK, ECHUNK)], ibs[r], isem[r])
        pltpu.async_copy(sdst.at[pl.ds(k * ECHUNK, ECHUNK)], ibd[r], isem[r])

    def _wait_idx(r):
        for _ in range(2):
            pltpu.make_async_copy(src_hbm.at[pl.ds(0, ECHUNK)], ibs[r],
                                  isem[r]).wait()

    _fire_idx(0, 0)

    fofs = [jnp.full((L,), f * NPAD, jnp.int32) for f in range(FPT)]
    pofs = [jnp.full((L,), p * NPAD, jnp.int32) for p in range(FPT // 2)]

    def chunk_body(k, c):
        for r in range(2):
            @pl.when(k % 2 == r)
            def _():
                _wait_idx(r)

                @pl.when(k + 1 < ENCH)
                def _():
                    _fire_idx(k + 1, 1 - r)

                def grp(g, cc):
                    vals = []
                    for u in range(4):
                        srcv = ibs[r][pl.ds((g * 4 + u) * L, L)]
                        dstv = ibd[r][pl.ds((g * 4 + u) * L, L)]
                        ws = [plsc.load_gather(hloc, [srcv + pofs[p]])
                              for p in range(FPT // 2)]
                        vals.append((dstv, ws))
                    for dstv, ws in vals:
                        for p in range(FPT // 2):
                            a, b = plsc.unpack(
                                plsc.bitcast(ws[p], jnp.bfloat16),
                                format=plsc.PackFormat.INTERLEAVED)
                            plsc.addupdate_scatter(
                                acc, [dstv + fofs[2 * p]], a)
                            plsc.addupdate_scatter(
                                acc, [dstv + fofs[2 * p + 1]], b)
                    return cc

                lax.fori_loop(0, NGRP // 4, grp, 0)
        return c

    lax.fori_loop(0, ENCH, chunk_body, 0)
    pltpu.sync_copy(acc, out_hbm.at[pl.ds(fbase, FPT * NPAD)])


def _stage1_body(degp, xt, w1t, dinv_out, h1t_out):
    deg = degp[0:1, :] + degp[1:2, :] + 1.0
    dinv = lax.rsqrt(deg)
    dinv_out[...] = jnp.broadcast_to(dinv, (8, NPAD))
    h1t_out[...] = jnp.dot(w1t[...], xt[...],
                           preferred_element_type=jnp.float32) * dinv


_stage1 = pl.pallas_call(
    _stage1_body,
    out_shape=[jax.ShapeDtypeStruct((8, NPAD), jnp.float32),
               jax.ShapeDtypeStruct((H, NPAD), jnp.float32)],
)


def _stage2_body(acct, h1t, dinv8, b1c, w2t, h2t_out):
    dinv = dinv8[0:1, :]
    z = (acct[...] + h1t[...]) * dinv + b1c[...]
    z = jnp.maximum(z, 0.0)
    h2t_out[...] = jnp.dot(w2t[...], z,
                           preferred_element_type=jnp.float32) * dinv


_stage2 = pl.pallas_call(
    _stage2_body,
    out_shape=jax.ShapeDtypeStruct((H, NPAD), jnp.float32),
)


def _stage3_body(acct, h2t, dinv8, b2c, batch8, wl, bl, out):
    z = (acct[...] + h2t[...]) * dinv8[0:1, :] + b2c[...]
    ids = batch8[0:1, :]
    seg = lax.broadcasted_iota(jnp.int32, (G, NPAD), 0)
    oht = (seg == ids).astype(jnp.float32)
    sums = lax.dot_general(oht, z, (((1,), (1,)), ((), ())),
                           preferred_element_type=jnp.float32)
    counts = jnp.sum(oht, axis=1, keepdims=True)
    pooled = sums / jnp.maximum(counts, 1.0)
    out[...] = jnp.dot(pooled, wl[...],
                       preferred_element_type=jnp.float32) + bl[...]


_stage3 = pl.pallas_call(
    _stage3_body,
    out_shape=jax.ShapeDtypeStruct((G, C), jnp.float32),
)


def kernel(x, edge_index, batch, W1, b1, W2, b2, Wl, bl):
    f32 = jnp.float32
    src = jnp.full((EPAD,), PAD_ROW, jnp.int32).at[:E].set(edge_index[0])
    dst = jnp.full((EPAD,), PAD_ROW, jnp.int32).at[:E].set(edge_index[1])
    dst_slab = dst.reshape(NW, DNCH, DCHUNK)
    xt = jnp.zeros((D, NPAD), f32).at[:, :N].set(x.T)
    bpad = jnp.pad(batch.astype(jnp.int32), (0, NPAD - N), constant_values=G)
    batch8 = jnp.broadcast_to(bpad[None, :], (8, NPAD))

    degp = _deg_kernel(dst_slab)
    degp8 = jnp.zeros((8, NPAD), f32).at[:2].set(degp.reshape(2, NPAD))

    def _pack(ht):
        hb = ht.astype(jnp.bfloat16).reshape(H // 2, 2, NPAD)
        hp = jax.lax.bitcast_convert_type(hb.transpose(0, 2, 1), f32)
        return hp.reshape(H // 2 * NPAD)

    dinv8, h1t = _stage1(degp8, xt, W1.T)
    acc1 = _edge_aggregate(_pack(h1t), src, dst).reshape(H, NPAD)
    h2t = _stage2(acc1, h1t, dinv8, b1.reshape(H, 1), W2.T)
    acc2 = _edge_aggregate(_pack(h2t), src, dst).reshape(H, NPAD)
    out = _stage3(acc2, h2t, dinv8, b2.reshape(H, 1), batch8,
                  Wl, bl.reshape(1, C))
    return out


# speedup vs baseline: 2.1519x; 1.0104x over previous
import functools

import jax
import jax.numpy as jnp
from jax import lax
from jax.experimental import pallas as pl
from jax.experimental.pallas import tpu as pltpu
from jax.experimental.pallas import tpu_sc as plsc

N = 10000
E = 320000
D = 128
H = 128
C = 10
G = 64

NC, NS, L = 2, 16, 16
NW = NC * NS
NPAD = 10240
FPT = H // NW
PAD_ROW = N

DCHUNK = 128
DNCH = 80
EPW = DNCH * DCHUNK
EPAD = NW * EPW

ECHUNK = 4096
ENCH = EPAD // ECHUNK
NGRP = ECHUNK // L
ESPT = EPAD // NS

_mesh = plsc.VectorSubcoreMesh(core_axis_name="c", subcore_axis_name="s",
                               num_cores=NC, num_subcores=NS)


@functools.partial(
    pl.kernel,
    out_type=jax.ShapeDtypeStruct((NC * NPAD,), jnp.float32),
    mesh=_mesh,
    scratch_types=[
        pltpu.VMEM((DNCH, DCHUNK), jnp.int32),
        pltpu.VMEM((DCHUNK,), jnp.float32),
        pltpu.VMEM((NPAD // NS,), jnp.float32),
        pltpu.SemaphoreType.DMA,
        pltpu.VMEM_SHARED((NPAD,), jnp.float32),
    ],
)
def _deg_kernel(dst_hbm, out_hbm, dsts, onesv, zv, dsem, acc):
    cid = lax.axis_index("c")
    sid = lax.axis_index("s")
    wid = cid * NS + sid
    rpt = NPAD // NS
    for j in range(DCHUNK // L):
        onesv[pl.ds(j * L, L)] = jnp.ones((L,), jnp.float32)

    def zb(i, c):
        zv[pl.ds(i * L, L)] = jnp.zeros((L,), jnp.float32)
        return c

    lax.fori_loop(0, rpt // L, zb, 0)
    pltpu.sync_copy(zv, acc.at[pl.ds(sid * rpt, rpt)])
    pltpu.sync_copy(dst_hbm.at[wid], dsts)
    plsc.subcore_barrier()

    def body(i, c):
        pltpu.async_copy(onesv, acc.at[dsts.at[i]], dsem, add=True)
        return c

    lax.fori_loop(0, DNCH, body, 0)

    def drain(i, c):
        pltpu.make_async_copy(out_hbm.at[pl.ds(0, DCHUNK)], onesv,
                              dsem).wait()
        return c

    lax.fori_loop(0, DNCH, drain, 0)
    plsc.subcore_barrier()
    pltpu.sync_copy(acc.at[pl.ds(sid * rpt, rpt)],
                    out_hbm.at[pl.ds(cid * NPAD + sid * rpt, rpt)])


@functools.partial(
    pl.kernel,
    out_type=jax.ShapeDtypeStruct((H * NPAD,), jnp.float32),
    mesh=_mesh,
    scratch_types=[
        pltpu.VMEM((FPT // 2 * NPAD,), jnp.float32),
        pltpu.VMEM((FPT * NPAD,), jnp.float32),
        [pltpu.VMEM((ECHUNK,), jnp.int32)] * 2,
        [pltpu.VMEM((ECHUNK,), jnp.int32)] * 2,
        [pltpu.SemaphoreType.DMA] * 2,
        pltpu.VMEM_SHARED((EPAD,), jnp.int32),
        pltpu.VMEM_SHARED((EPAD,), jnp.int32),
    ],
    compiler_params=pltpu.CompilerParams(needs_layout_passes=False),
)
def _edge_aggregate(ht_hbm, src_hbm, dst_hbm, out_hbm,
                    hloc, acc, ibs, ibd, isem, ssrc, sdst):
    cid = lax.axis_index("c")
    sid = lax.axis_index("s")
    wid = cid * NS + sid
    fbase = wid * FPT * NPAD
    pbase = wid * (FPT // 2) * NPAD

    pltpu.sync_copy(src_hbm.at[pl.ds(sid * ESPT, ESPT)],
                    ssrc.at[pl.ds(sid * ESPT, ESPT)])
    pltpu.sync_copy(dst_hbm.at[pl.ds(sid * ESPT, ESPT)],
                    sdst.at[pl.ds(sid * ESPT, ESPT)])
    pltpu.sync_copy(ht_hbm.at[pl.ds(pbase, FPT // 2 * NPAD)], hloc)

    def zb(i, c):
        acc[pl.ds(i * L, L)] = jnp.zeros((L,), jnp.float32)
        return c

    lax.fori_loop(0, FPT * NPAD // L, zb, 0)
    plsc.subcore_barrier()

    def _fire_idx(k, r):
        pltpu.async_copy(ssrc.at[pl.ds(k * ECHUNK, ECHUNK)], ibs[r], isem[r])
        pltpu.async_copy(sdst.at[pl.ds(k * ECHUNK, ECHUNK)], ibd[r], isem[r])

    def _wait_idx(r):
        for _ in range(2):
            pltpu.make_async_copy(src_hbm.at[pl.ds(0, ECHUNK)], ibs[r],
                                  isem[r]).wait()

    _fire_idx(0, 0)

    fofs = [jnp.full((L,), f * NPAD, jnp.int32) for f in range(FPT)]
    pofs = [jnp.full((L,), p * NPAD, jnp.int32) for p in range(FPT // 2)]

    def chunk_body(k, c):
        for r in range(2):
            @pl.when(k % 2 == r)
            def _():
                _wait_idx(r)

                @pl.when(k + 1 < ENCH)
                def _():
                    _fire_idx(k + 1, 1 - r)

                def grp(g, cc):
                    vals = []
                    for u in range(4):
                        srcv = ibs[r][pl.ds((g * 4 + u) * L, L)]
                        dstv = ibd[r][pl.ds((g * 4 + u) * L, L)]
                        ws = [plsc.load_gather(hloc, [srcv + pofs[p]])
                              for p in range(FPT // 2)]
                        vals.append((dstv, ws))
                    for dstv, ws in vals:
                        for p in range(FPT // 2):
                            a, b = plsc.unpack(
                                plsc.bitcast(ws[p], jnp.bfloat16),
                                format=plsc.PackFormat.INTERLEAVED)
                            plsc.addupdate_scatter(
                                acc, [dstv + fofs[2 * p]], a)
                            plsc.addupdate_scatter(
                                acc, [dstv + fofs[2 * p + 1]], b)
                    return cc

                lax.fori_loop(0, NGRP // 4, grp, 0)
        return c

    lax.fori_loop(0, ENCH, chunk_body, 0)
    pltpu.sync_copy(acc, out_hbm.at[pl.ds(fbase, FPT * NPAD)])


def _stage1_body(degp, xt, w1t, dinv_out, h1t_out):
    deg = degp[0:1, :] + degp[1:2, :] + 1.0
    dinv = lax.rsqrt(deg)
    dinv_out[...] = jnp.broadcast_to(dinv, (8, NPAD))
    h1t_out[...] = jnp.dot(w1t[...], xt[...],
                           preferred_element_type=jnp.float32) * dinv


_stage1 = pl.pallas_call(
    _stage1_body,
    out_shape=[jax.ShapeDtypeStruct((8, NPAD), jnp.float32),
               jax.ShapeDtypeStruct((H, NPAD), jnp.float32)],
)


def _stage2_body(acct, h1t, dinv8, b1c, w2t, h2t_out):
    dinv = dinv8[0:1, :]
    z = (acct[...] + h1t[...]) * dinv + b1c[...]
    z = jnp.maximum(z, 0.0)
    h2t_out[...] = jnp.dot(w2t[...], z,
                           preferred_element_type=jnp.float32) * dinv


_stage2 = pl.pallas_call(
    _stage2_body,
    out_shape=jax.ShapeDtypeStruct((H, NPAD), jnp.float32),
)


def _stage3_body(acct, h2t, dinv8, b2c, batch8, wl, bl, out):
    z = (acct[...] + h2t[...]) * dinv8[0:1, :] + b2c[...]
    ids = batch8[0:1, :]
    seg = lax.broadcasted_iota(jnp.int32, (G, NPAD), 0)
    oht = (seg == ids).astype(jnp.float32)
    sums = lax.dot_general(oht, z, (((1,), (1,)), ((), ())),
                           preferred_element_type=jnp.float32)
    counts = jnp.sum(oht, axis=1, keepdims=True)
    pooled = sums / jnp.maximum(counts, 1.0)
    out[...] = jnp.dot(pooled, wl[...],
                       preferred_element_type=jnp.float32) + bl[...]


_stage3 = pl.pallas_call(
    _stage3_body,
    out_shape=jax.ShapeDtypeStruct((G, C), jnp.float32),
)


def kernel(x, edge_index, batch, W1, b1, W2, b2, Wl, bl):
    f32 = jnp.float32
    src = jnp.full((EPAD,), PAD_ROW, jnp.int32).at[:E].set(edge_index[0])
    dst = jnp.full((EPAD,), PAD_ROW, jnp.int32).at[:E].set(edge_index[1])
    dst_slab = dst.reshape(NW, DNCH, DCHUNK)
    xt = jnp.zeros((D, NPAD), f32).at[:, :N].set(x.T)
    bpad = jnp.pad(batch.astype(jnp.int32), (0, NPAD - N), constant_values=G)
    batch8 = jnp.broadcast_to(bpad[None, :], (8, NPAD))

    degp = _deg_kernel(dst_slab)
    degp8 = jnp.zeros((8, NPAD), f32).at[:2].set(degp.reshape(2, NPAD))

    def _pack(ht):
        hb = ht.astype(jnp.bfloat16).reshape(H // 2, 2, NPAD)
        hp = jax.lax.bitcast_convert_type(hb.transpose(0, 2, 1), f32)
        return hp.reshape(H // 2 * NPAD)

    dinv8, h1t = _stage1(degp8, xt, W1.T)
    acc1 = _edge_aggregate(_pack(h1t), src, dst).reshape(H, NPAD)
    h2t = _stage2(acc1, h1t, dinv8, b1.reshape(H, 1), W2.T)
    acc2 = _edge_aggregate(_pack(h2t), src, dst).reshape(H, NPAD)
    out = _stage3(acc2, h2t, dinv8, b2.reshape(H, 1), batch8,
                  Wl, bl.reshape(1, C))
    return out
